# Initial kernel scaffold; baseline (speedup 1.0000x reference)
import jax
import jax.numpy as jnp
from jax.experimental import pallas as pl


def kernel(initial_features, edge_index, edge_weight, homophily_bases, comb_weight, h_hat_for_view, tau_for_view):
    raise NotImplementedError("write your pallas kernel here")


# speedup vs baseline: 3.7462x; 3.7462x over previous
import functools
import math

import jax
import jax.numpy as jnp
from jax import lax
from jax.experimental import pallas as pl
from jax.experimental.pallas import tpu as pltpu
from jax.experimental.pallas import tpu_sc as plsc

N = 10000
N_PAD = 10240
F = 128
NC = 2
NS = 16
NW = NC * NS
CH = 128
CHUNKS_PER_W = 80
E_PAD = NW * CH * CHUNKS_PER_W
ROWS_PER_TILE = N_PAD // NS


def _build_sc_spmm():
    mesh = plsc.VectorSubcoreMesh(core_axis_name="c", subcore_axis_name="s")

    @functools.partial(
        pl.kernel,
        out_type=jax.ShapeDtypeStruct((NC, N_PAD, F), jnp.float32),
        mesh=mesh,
        scratch_types=[
            pltpu.VMEM((CH,), jnp.int32),
            pltpu.VMEM((CH,), jnp.int32),
            pltpu.VMEM((CH // 8, F), jnp.float32),
            pltpu.VMEM((CH, F), jnp.float32),
            pltpu.VMEM_SHARED((N_PAD, F), jnp.float32),
            pltpu.SemaphoreType.DMA,
        ],
    )
    def spmm(src_hbm, dst_hbm, w_hbm, u_hbm, zeros_hbm, out_hbm,
             srcv, dstv, wv, rows, accsh, sem):
        cid = lax.axis_index("c")
        sid = lax.axis_index("s")
        pltpu.sync_copy(
            zeros_hbm.at[pl.ds(sid * ROWS_PER_TILE, ROWS_PER_TILE)],
            accsh.at[pl.ds(sid * ROWS_PER_TILE, ROWS_PER_TILE)])
        plsc.subcore_barrier()

        wid = cid * NS + sid
        base = wid * (CHUNKS_PER_W * CH)

        def chunk(i, carry):
            off = base + i * CH
            pltpu.sync_copy(src_hbm.at[pl.ds(off, CH)], srcv)
            pltpu.sync_copy(dst_hbm.at[pl.ds(off, CH)], dstv)
            woff = pl.multiple_of(
                wid * (CHUNKS_PER_W * CH // 8) + i * (CH // 8), 8)
            pltpu.sync_copy(w_hbm.at[pl.ds(woff, CH // 8)], wv)
            pltpu.async_copy(u_hbm.at[srcv], rows, sem).wait()

            def scale8(g, c2):
                for u in range(8):
                    e = g * 8 + u
                    wsplat = wv[g, pl.ds(u * 16, 16)]
                    for j in range(F // 16):
                        sl = pl.ds(j * 16, 16)
                        rows[e, sl] = rows[e, sl] * wsplat
                return c2

            lax.fori_loop(0, CH // 8, scale8, 0)
            pltpu.sync_copy(rows, accsh.at[dstv], add=True)
            return carry

        lax.fori_loop(0, CHUNKS_PER_W, chunk, 0)
        plsc.subcore_barrier()
        pltpu.sync_copy(
            accsh.at[pl.ds(sid * ROWS_PER_TILE, ROWS_PER_TILE)],
            out_hbm.at[cid, pl.ds(sid * ROWS_PER_TILE, ROWS_PER_TILE)])

    return spmm


_sc_spmm_cache = []


def _sc_spmm(*args):
    if not _sc_spmm_cache:
        _sc_spmm_cache.append(_build_sc_spmm())
    return _sc_spmm_cache[0](*args)


def _tc_init_body(x_ref, u0_ref):
    x = x_ref[...]
    nrm = jnp.sqrt(jnp.sum(x * x, axis=0, keepdims=True))
    u0 = x / (nrm + 1e-8)
    u0_ref[...] = jnp.concatenate(
        [u0, jnp.zeros((N_PAD - N, F), jnp.float32)], axis=0)


_tc_init = pl.pallas_call(
    _tc_init_body,
    out_shape=jax.ShapeDtypeStruct((N_PAD, F), jnp.float32),
)


def _tc_hb_body(scal_ref, hb_ref, u0_ref, acc_ref):
    k = pl.program_id(0)
    hbk = hb_ref[0]
    nrm = jnp.sqrt(jnp.sum(hbk * hbk, axis=0, keepdims=True))
    contrib = scal_ref[k] * (hbk / (nrm + 1e-8))
    contrib = jnp.concatenate(
        [contrib, jnp.zeros((N_PAD - N, F), jnp.float32)], axis=0)

    @pl.when(k == 0)
    def _():
        acc_ref[...] = scal_ref[11] * u0_ref[...] + contrib

    @pl.when(k > 0)
    def _():
        acc_ref[...] = acc_ref[...] + contrib


_tc_hb = pl.pallas_call(
    _tc_hb_body,
    grid=(11,),
    in_specs=[
        pl.BlockSpec(memory_space=pltpu.SMEM),
        pl.BlockSpec((1, N, F), lambda k: (k, 0, 0)),
        pl.BlockSpec((N_PAD, F), lambda k: (0, 0)),
    ],
    out_specs=pl.BlockSpec((N_PAD, F), lambda k: (0, 0)),
    out_shape=jax.ShapeDtypeStruct((N_PAD, F), jnp.float32),
)


def _tc_hop_body(scal_ref, vr_ref, up_ref, upp_ref, s_ref, acc_ref,
                 uk_ref, snew_ref, accnew_ref):
    ct = scal_ref[0]
    st = scal_ref[1]
    coeff = scal_ref[2]
    v = vr_ref[0] + vr_ref[1]
    up = up_ref[...]
    upp = upp_ref[...]
    v = v - jnp.sum(v * up, axis=0, keepdims=True) * up
    v = v - jnp.sum(v * upp, axis=0, keepdims=True) * upp
    v = v / (jnp.sqrt(jnp.sum(v * v, axis=0, keepdims=True)) + 1e-8)
    s = s_ref[...]
    sn = s / (jnp.sqrt(jnp.sum(s * s, axis=0, keepdims=True)) + 1e-8)
    t = ct * sn + st * v
    u_k = t / (jnp.sqrt(jnp.sum(t * t, axis=0, keepdims=True)) + 1e-8)
    uk_ref[...] = u_k
    snew_ref[...] = s + u_k
    accnew_ref[...] = acc_ref[...] + coeff * u_k


_tc_hop = pl.pallas_call(
    _tc_hop_body,
    in_specs=[
        pl.BlockSpec(memory_space=pltpu.SMEM),
        pl.BlockSpec(memory_space=pltpu.VMEM),
        pl.BlockSpec(memory_space=pltpu.VMEM),
        pl.BlockSpec(memory_space=pltpu.VMEM),
        pl.BlockSpec(memory_space=pltpu.VMEM),
        pl.BlockSpec(memory_space=pltpu.VMEM),
    ],
    out_shape=[
        jax.ShapeDtypeStruct((N_PAD, F), jnp.float32),
        jax.ShapeDtypeStruct((N_PAD, F), jnp.float32),
        jax.ShapeDtypeStruct((N_PAD, F), jnp.float32),
    ],
)


def kernel(initial_features, edge_index, edge_weight, homophily_bases,
           comb_weight, h_hat_for_view, tau_for_view):
    k_hops = homophily_bases.shape[0] - 1
    theta = jnp.asarray((math.pi / 2.0) * (1.0 - h_hat_for_view), jnp.float32)
    ct = jnp.cos(theta)
    st = jnp.sin(theta)
    tau = jnp.asarray(tau_for_view, jnp.float32)
    coeffs = comb_weight[0, :, 0].astype(jnp.float32)

    src = edge_index[0].astype(jnp.int32)
    dst = edge_index[1].astype(jnp.int32)
    w = edge_weight.astype(jnp.float32)
    e = src.shape[0]
    pad = E_PAD - e
    pad_idx = (jnp.arange(pad, dtype=jnp.int32) * 131) % N
    src_p = jnp.concatenate([src, pad_idx])
    dst_p = jnp.concatenate([dst, pad_idx])
    w_p = jnp.concatenate([w, jnp.zeros((pad,), jnp.float32)])
    w_exp = jnp.repeat(w_p, 16).reshape(E_PAD // 8, F)

    zeros_nf = jnp.zeros((N_PAD, F), jnp.float32)

    u0 = _tc_init(initial_features)
    scal_hb = jnp.concatenate([tau * coeffs, ((1.0 - tau) * coeffs[0])[None]])
    acc = _tc_hb(scal_hb, homophily_bases, u0)

    s = u0
    u_p = u0
    u_pp = zeros_nf
    for k in range(1, k_hops + 1):
        vparts = _sc_spmm(src_p, dst_p, w_exp, u_p, zeros_nf)
        scal = jnp.stack([ct, st, (1.0 - tau) * coeffs[k]])
        u_k, s, acc = _tc_hop(scal, vparts, u_p, u_pp, s, acc)
        u_pp, u_p = u_p, u_k
    return acc[:N]


# speedup vs baseline: 6.3836x; 1.7040x over previous
import functools
import math

import jax
import jax.numpy as jnp
from jax import lax
from jax.experimental import pallas as pl
from jax.experimental.pallas import tpu as pltpu
from jax.experimental.pallas import tpu_sc as plsc

N = 10000
N_PAD = 10240
F = 128
NC = 2
NS = 16
NW = NC * NS
CH = 128
CHUNKS_PER_W = 80
E_PAD = NW * CH * CHUNKS_PER_W
ROWS_PER_TILE = N_PAD // NS


def _build_sc_spmm():
    mesh = plsc.VectorSubcoreMesh(core_axis_name="c", subcore_axis_name="s")

    @functools.partial(
        pl.kernel,
        out_type=jax.ShapeDtypeStruct((NC, N_PAD, F), jnp.float32),
        mesh=mesh,
        scratch_types=[
            pltpu.VMEM((2, CH), jnp.int32),
            pltpu.VMEM((2, CH), jnp.int32),
            pltpu.VMEM((2, CH // 8, F), jnp.float32),
            pltpu.VMEM((2, CH, F), jnp.float32),
            pltpu.VMEM_SHARED((N_PAD, F), jnp.float32),
            pltpu.SemaphoreType.DMA,
            pltpu.SemaphoreType.DMA,
            pltpu.SemaphoreType.DMA,
            pltpu.SemaphoreType.DMA,
        ],
    )
    def spmm(src_hbm, dst_hbm, w_hbm, u_hbm, zeros_hbm, out_hbm,
             srcv, dstv, wv, rows, accsh, semg0, semg1, semi0, semi1):
        cid = lax.axis_index("c")
        sid = lax.axis_index("s")
        semg = (semg0, semg1)
        semi = (semi0, semi1)

        wid = cid * NS + sid
        base = wid * (CHUNKS_PER_W * CH)
        wbase = wid * (CHUNKS_PER_W * (CH // 8))

        def idx_copies(i, b):
            off = base + i * CH
            woff = pl.multiple_of(wbase + i * (CH // 8), 8)
            return (
                (src_hbm.at[pl.ds(off, CH)], srcv.at[b], semi[b]),
                (dst_hbm.at[pl.ds(off, CH)], dstv.at[b], semi[b]),
                (w_hbm.at[pl.ds(woff, CH // 8)], wv.at[b], semi[b]),
            )

        def idx_issue(i, b):
            for args in idx_copies(i, b):
                pltpu.async_copy(*args)

        def idx_wait(i, b):
            for args in idx_copies(i, b):
                pltpu.make_async_copy(*args).wait()

        def gather_issue(b):
            pltpu.async_copy(u_hbm.at[srcv.at[b]], rows.at[b], semg[b])

        def gather_wait(b):
            pltpu.make_async_copy(
                u_hbm.at[srcv.at[b]], rows.at[b], semg[b]).wait()

        def scale_scatter(b):
            def scale8(g, c2):
                for u in range(8):
                    e = g * 8 + u
                    wsplat = wv[b, g, pl.ds(u * 16, 16)]
                    for j in range(F // 16):
                        sl = pl.ds(j * 16, 16)
                        rows[b, e, sl] = rows[b, e, sl] * wsplat
                return c2

            lax.fori_loop(0, CH // 8, scale8, 0)
            pltpu.sync_copy(rows.at[b], accsh.at[dstv.at[b]], add=True)

        for args in idx_copies(0, 0):
            pltpu.async_copy(*args).wait()
        gather_issue(0)
        idx_issue(1, 1)

        pltpu.sync_copy(
            zeros_hbm.at[pl.ds(sid * ROWS_PER_TILE, ROWS_PER_TILE)],
            accsh.at[pl.ds(sid * ROWS_PER_TILE, ROWS_PER_TILE)])
        plsc.subcore_barrier()

        def step(i, b, issue_gather, issue_idx):
            if issue_gather:
                idx_wait(i + 1, 1 - b)
                gather_issue(1 - b)
            gather_wait(b)
            scale_scatter(b)
            if issue_idx:
                idx_issue(i + 2, b)

        def pair(g, carry):
            step(2 * g, 0, True, True)
            step(2 * g + 1, 1, True, True)
            return carry

        lax.fori_loop(0, (CHUNKS_PER_W - 2) // 2, pair, 0)
        step(CHUNKS_PER_W - 2, 0, True, False)
        step(CHUNKS_PER_W - 1, 1, False, False)
        plsc.subcore_barrier()
        pltpu.sync_copy(
            accsh.at[pl.ds(sid * ROWS_PER_TILE, ROWS_PER_TILE)],
            out_hbm.at[cid, pl.ds(sid * ROWS_PER_TILE, ROWS_PER_TILE)])

    return spmm


_sc_spmm_cache = []


def _sc_spmm(*args):
    if not _sc_spmm_cache:
        _sc_spmm_cache.append(_build_sc_spmm())
    return _sc_spmm_cache[0](*args)


def _tc_init_body(x_ref, u0_ref):
    x = x_ref[...]
    nrm = jnp.sqrt(jnp.sum(x * x, axis=0, keepdims=True))
    u0 = x / (nrm + 1e-8)
    u0_ref[...] = jnp.concatenate(
        [u0, jnp.zeros((N_PAD - N, F), jnp.float32)], axis=0)


_tc_init = pl.pallas_call(
    _tc_init_body,
    out_shape=jax.ShapeDtypeStruct((N_PAD, F), jnp.float32),
)


def _tc_hb_body(scal_ref, hb_ref, u0_ref, acc_ref):
    k = pl.program_id(0)
    hbk = hb_ref[0]
    nrm = jnp.sqrt(jnp.sum(hbk * hbk, axis=0, keepdims=True))
    contrib = scal_ref[k] * (hbk / (nrm + 1e-8))
    contrib = jnp.concatenate(
        [contrib, jnp.zeros((N_PAD - N, F), jnp.float32)], axis=0)

    @pl.when(k == 0)
    def _():
        acc_ref[...] = scal_ref[11] * u0_ref[...] + contrib

    @pl.when(k > 0)
    def _():
        acc_ref[...] = acc_ref[...] + contrib


_tc_hb = pl.pallas_call(
    _tc_hb_body,
    grid=(11,),
    in_specs=[
        pl.BlockSpec(memory_space=pltpu.SMEM),
        pl.BlockSpec((1, N, F), lambda k: (k, 0, 0)),
        pl.BlockSpec((N_PAD, F), lambda k: (0, 0)),
    ],
    out_specs=pl.BlockSpec((N_PAD, F), lambda k: (0, 0)),
    out_shape=jax.ShapeDtypeStruct((N_PAD, F), jnp.float32),
)


def _tc_hop_body(scal_ref, vr_ref, up_ref, upp_ref, s_ref, acc_ref,
                 uk_ref, snew_ref, accnew_ref):
    ct = scal_ref[0]
    st = scal_ref[1]
    coeff = scal_ref[2]
    v = vr_ref[0] + vr_ref[1]
    up = up_ref[...]
    upp = upp_ref[...]
    v = v - jnp.sum(v * up, axis=0, keepdims=True) * up
    v = v - jnp.sum(v * upp, axis=0, keepdims=True) * upp
    v = v / (jnp.sqrt(jnp.sum(v * v, axis=0, keepdims=True)) + 1e-8)
    s = s_ref[...]
    sn = s / (jnp.sqrt(jnp.sum(s * s, axis=0, keepdims=True)) + 1e-8)
    t = ct * sn + st * v
    u_k = t / (jnp.sqrt(jnp.sum(t * t, axis=0, keepdims=True)) + 1e-8)
    uk_ref[...] = u_k
    snew_ref[...] = s + u_k
    accnew_ref[...] = acc_ref[...] + coeff * u_k


_tc_hop = pl.pallas_call(
    _tc_hop_body,
    in_specs=[
        pl.BlockSpec(memory_space=pltpu.SMEM),
        pl.BlockSpec(memory_space=pltpu.VMEM),
        pl.BlockSpec(memory_space=pltpu.VMEM),
        pl.BlockSpec(memory_space=pltpu.VMEM),
        pl.BlockSpec(memory_space=pltpu.VMEM),
        pl.BlockSpec(memory_space=pltpu.VMEM),
    ],
    out_shape=[
        jax.ShapeDtypeStruct((N_PAD, F), jnp.float32),
        jax.ShapeDtypeStruct((N_PAD, F), jnp.float32),
        jax.ShapeDtypeStruct((N_PAD, F), jnp.float32),
    ],
)


def kernel(initial_features, edge_index, edge_weight, homophily_bases,
           comb_weight, h_hat_for_view, tau_for_view):
    k_hops = homophily_bases.shape[0] - 1
    theta = jnp.asarray((math.pi / 2.0) * (1.0 - h_hat_for_view), jnp.float32)
    ct = jnp.cos(theta)
    st = jnp.sin(theta)
    tau = jnp.asarray(tau_for_view, jnp.float32)
    coeffs = comb_weight[0, :, 0].astype(jnp.float32)

    src = edge_index[0].astype(jnp.int32)
    dst = edge_index[1].astype(jnp.int32)
    w = edge_weight.astype(jnp.float32)
    e = src.shape[0]
    pad = E_PAD - e
    pad_idx = (jnp.arange(pad, dtype=jnp.int32) * 131) % N
    src_p = jnp.concatenate([src, pad_idx])
    dst_p = jnp.concatenate([dst, pad_idx])
    w_p = jnp.concatenate([w, jnp.zeros((pad,), jnp.float32)])
    w_exp = jnp.repeat(w_p, 16).reshape(E_PAD // 8, F)

    zeros_nf = jnp.zeros((N_PAD, F), jnp.float32)

    u0 = _tc_init(initial_features)
    scal_hb = jnp.concatenate([tau * coeffs, ((1.0 - tau) * coeffs[0])[None]])
    acc = _tc_hb(scal_hb, homophily_bases, u0)

    s = u0
    u_p = u0
    u_pp = zeros_nf
    for k in range(1, k_hops + 1):
        vparts = _sc_spmm(src_p, dst_p, w_exp, u_p, zeros_nf)
        scal = jnp.stack([ct, st, (1.0 - tau) * coeffs[k]])
        u_k, s, acc = _tc_hop(scal, vparts, u_p, u_pp, s, acc)
        u_pp, u_p = u_p, u_k
    return acc[:N]
